# baseline (device time: 139309 ns/iter reference)
import functools

import jax
import jax.numpy as jnp
from jax import lax
from jax.experimental import pallas as pl
from jax.experimental.pallas import tpu as pltpu

N_DEV = 4
TILE = 512


def _layer_body(x_ref, win_ref, wout_ref, out_ref,
                comm_ref, send_sems, recv_sems, *, n_steps):
    j = pl.program_id(0)

    h = jnp.maximum(
        jnp.dot(x_ref[...], win_ref[...], preferred_element_type=jnp.float32),
        0.0,
    )
    contrib = jnp.dot(h, wout_ref[...], preferred_element_type=jnp.float32)

    @pl.when(j == 0)
    def _():
        out_ref[...] = contrib

    @pl.when(j > 0)
    def _():
        out_ref[...] = out_ref[...] + contrib

    @pl.when(j == n_steps - 1)
    def _():
        my_pos = lax.axis_index("i")
        left = (my_pos + N_DEV - 1) % N_DEV
        right = (my_pos + 1) % N_DEV

        barrier = pltpu.get_barrier_semaphore()
        for nbr in (left, right):
            pl.semaphore_signal(
                barrier, inc=1,
                device_id=(nbr,), device_id_type=pl.DeviceIdType.MESH,
            )
        pl.semaphore_wait(barrier, 2)

        comm_ref[0, :, :] = out_ref[...]
        for h_ in range(N_DEV - 1):
            rdma = pltpu.make_async_remote_copy(
                src_ref=comm_ref.at[h_],
                dst_ref=comm_ref.at[h_ + 1],
                send_sem=send_sems.at[h_],
                recv_sem=recv_sems.at[h_],
                device_id=(right,),
                device_id_type=pl.DeviceIdType.MESH,
            )
            rdma.start()
            rdma.wait()
            out_ref[...] = out_ref[...] + comm_ref[h_ + 1, :, :]


def _layer(x, win, wout, cid):
    m, d_in = x.shape
    d_hid = win.shape[1]
    d_out = wout.shape[1]
    n_steps = d_hid // TILE

    return pl.pallas_call(
        functools.partial(_layer_body, n_steps=n_steps),
        grid=(n_steps,),
        out_shape=jax.ShapeDtypeStruct((m, d_out), jnp.float32),
        in_specs=[
            pl.BlockSpec((m, d_in), lambda j: (0, 0)),
            pl.BlockSpec((d_in, TILE), lambda j: (0, j)),
            pl.BlockSpec((TILE, d_out), lambda j: (j, 0)),
        ],
        out_specs=pl.BlockSpec((m, d_out), lambda j: (0, 0)),
        scratch_shapes=[
            pltpu.VMEM((N_DEV, m, d_out), jnp.float32),
            pltpu.SemaphoreType.DMA((N_DEV - 1,)),
            pltpu.SemaphoreType.DMA((N_DEV - 1,)),
        ],
        compiler_params=pltpu.CompilerParams(
            dimension_semantics=("arbitrary",),
            collective_id=cid,
        ),
    )(x, win, wout)


def kernel(x, Win0, Wout0, Win1, Wout1, Win2, Wout2):
    x = _layer(x, Win0, Wout0, 0)
    x = _layer(x, Win1, Wout1, 1)
    x = _layer(x, Win2, Wout2, 2)
    return x


# device time: 99783 ns/iter; 1.3961x vs baseline; 1.3961x over previous
import functools

import jax
import jax.numpy as jnp
from jax import lax
from jax.experimental import pallas as pl
from jax.experimental.pallas import tpu as pltpu

N_DEV = 4
TILE = 512


def _layer_body(x_ref, win_ref, wout_ref, out_ref,
                comm_ref, send_sems, recv_sems, *, n_steps):
    j = pl.program_id(0)

    h = jnp.maximum(
        jnp.dot(x_ref[...], win_ref[...], preferred_element_type=jnp.float32),
        0.0,
    )
    contrib = jnp.dot(h, wout_ref[...], preferred_element_type=jnp.float32)

    @pl.when(j == 0)
    def _():
        out_ref[...] = contrib

    @pl.when(j > 0)
    def _():
        out_ref[...] = out_ref[...] + contrib

    @pl.when(j == n_steps - 1)
    def _():
        my_pos = lax.axis_index("i")
        left = (my_pos + N_DEV - 1) % N_DEV
        right = (my_pos + 1) % N_DEV

        barrier = pltpu.get_barrier_semaphore()
        for nbr in (left, right):
            pl.semaphore_signal(
                barrier, inc=1,
                device_id=(nbr,), device_id_type=pl.DeviceIdType.MESH,
            )
        pl.semaphore_wait(barrier, 2)

        half = out_ref.shape[1] // 2
        p_xor = my_pos ^ 1
        p_rev = 3 - my_pos
        for r in range(2):
            partner_a = p_xor if r == 0 else p_rev
            partner_b = p_rev if r == 0 else p_xor
            rdma_a = pltpu.make_async_remote_copy(
                src_ref=out_ref.at[:, pl.ds(0, half)],
                dst_ref=comm_ref.at[2 * r],
                send_sem=send_sems.at[2 * r],
                recv_sem=recv_sems.at[2 * r],
                device_id=(partner_a,),
                device_id_type=pl.DeviceIdType.MESH,
            )
            rdma_b = pltpu.make_async_remote_copy(
                src_ref=out_ref.at[:, pl.ds(half, half)],
                dst_ref=comm_ref.at[2 * r + 1],
                send_sem=send_sems.at[2 * r + 1],
                recv_sem=recv_sems.at[2 * r + 1],
                device_id=(partner_b,),
                device_id_type=pl.DeviceIdType.MESH,
            )
            rdma_a.start()
            rdma_b.start()
            rdma_a.wait()
            rdma_b.wait()
            out_ref[:, pl.ds(0, half)] += comm_ref[2 * r, :, :]
            out_ref[:, pl.ds(half, half)] += comm_ref[2 * r + 1, :, :]


def _layer(x, win, wout, cid):
    m, d_in = x.shape
    d_hid = win.shape[1]
    d_out = wout.shape[1]
    n_steps = d_hid // TILE

    return pl.pallas_call(
        functools.partial(_layer_body, n_steps=n_steps),
        grid=(n_steps,),
        out_shape=jax.ShapeDtypeStruct((m, d_out), jnp.float32),
        in_specs=[
            pl.BlockSpec((m, d_in), lambda j: (0, 0)),
            pl.BlockSpec((d_in, TILE), lambda j: (0, j)),
            pl.BlockSpec((TILE, d_out), lambda j: (j, 0)),
        ],
        out_specs=pl.BlockSpec((m, d_out), lambda j: (0, 0)),
        scratch_shapes=[
            pltpu.VMEM((4, m, d_out // 2), jnp.float32),
            pltpu.SemaphoreType.DMA((4,)),
            pltpu.SemaphoreType.DMA((4,)),
        ],
        compiler_params=pltpu.CompilerParams(
            dimension_semantics=("arbitrary",),
            collective_id=cid,
        ),
    )(x, win, wout)


def kernel(x, Win0, Wout0, Win1, Wout1, Win2, Wout2):
    x = _layer(x, Win0, Wout0, 0)
    x = _layer(x, Win1, Wout1, 1)
    x = _layer(x, Win2, Wout2, 2)
    return x
